# Initial kernel scaffold; baseline (speedup 1.0000x reference)
import jax
import jax.numpy as jnp
from jax.experimental import pallas as pl


def kernel(origins, directions, lengths, densities, features, world2local):
    raise NotImplementedError("write your pallas kernel here")


# speedup vs baseline: 1.2603x; 1.2603x over previous
import functools

import jax
import jax.numpy as jnp
from jax import lax
from jax.experimental import pallas as pl
from jax.experimental.pallas import tpu as pltpu
from jax.experimental.pallas import tpu_sc as plsc

B, NR, P = 4, 2048, 64
D = H = W = 128
DHW = D * H * W
N = B * NR * P
NC, NS = 2, 16
NW = NC * NS
PPW = N // NW
CHK = 128
NCHUNK = PPW // CHK
NCORN = 8
LANES = 16
NGRP = CHK // LANES


def _interp_body(tab, px, py, pz, dens_out, feat_out,
                 px_v, py_v, pz_v, idx_v, w_v, rows_v, dens_v, feat_v, sem):
  wid = lax.axis_index("s") * NC + lax.axis_index("c")
  b = wid // (NW // B)
  row_base = b * DHW
  lane = lax.broadcasted_iota(jnp.int32, (LANES,), 0)

  def axis_prep(coord):
    f = (coord + 1.0) * ((D - 1) * 0.5)
    t = f.astype(jnp.int32)
    i0 = t - jnp.where(t.astype(jnp.float32) > f, 1, 0)
    w1 = f - i0.astype(jnp.float32)
    w0 = 1.0 - w1
    i1 = i0 + 1
    w0m = jnp.where((i0 >= 0) & (i0 < D), w0, 0.0)
    w1m = jnp.where((i1 >= 0) & (i1 < D), w1, 0.0)
    return jnp.clip(i0, 0, D - 1), jnp.clip(i1, 0, D - 1), w0m, w1m

  def chunk(k, carry):
    base = wid * PPW + k * CHK
    pltpu.sync_copy(px.at[pl.ds(base, CHK)], px_v)
    pltpu.sync_copy(py.at[pl.ds(base, CHK)], py_v)
    pltpu.sync_copy(pz.at[pl.ds(base, CHK)], pz_v)

    def build(g, c2):
      sl = pl.ds(g * LANES, LANES)
      x0, x1, wx0, wx1 = axis_prep(px_v[sl])
      y0, y1, wy0, wy1 = axis_prep(py_v[sl])
      z0, z1, wz0, wz1 = axis_prep(pz_v[sl])
      c = 0
      for zc, wz in ((z0, wz0), (z1, wz1)):
        for yc, wy in ((y0, wy0), (y1, wy1)):
          for xc, wx in ((x0, wx0), (x1, wx1)):
            idx_v[c, sl] = row_base + (zc * H + yc) * W + xc
            w_v[c, sl] = wz * wy * wx
            c += 1
      return c2
    lax.fori_loop(0, NGRP, build, 0)

    copies = [pltpu.async_copy(tab.at[idx_v.at[c]],
                               rows_v.at[pl.ds(c * CHK, CHK)], sem)
              for c in range(NCORN)]
    for cp in copies:
      cp.wait()

    def interp(g, c2):
      sl = pl.ds(g * LANES, LANES)
      p_vec = g * LANES + lane
      for ch in range(9):
        ch_vec = jnp.full((LANES,), ch, jnp.int32)
        acc = jnp.zeros((LANES,), jnp.float32)
        for c in range(NCORN):
          val = plsc.load_gather(rows_v, [p_vec + c * CHK, ch_vec])
          acc = acc + val * w_v[c, sl]
        if ch == 0:
          dens_v[sl] = acc
        else:
          plsc.store_scatter(feat_v, [(g * LANES + lane) * 8 + (ch - 1)], acc)
      return c2
    lax.fori_loop(0, NGRP, interp, 0)

    pltpu.sync_copy(dens_v, dens_out.at[pl.ds(base, CHK)])
    pltpu.sync_copy(feat_v, feat_out.at[pl.ds(base * 8, CHK * 8)])
    return carry

  lax.fori_loop(0, NCHUNK, chunk, 0)


def kernel(origins, directions, lengths, densities, features, world2local):
  ones = jnp.ones(origins.shape[:-1] + (1,), dtype=origins.dtype)
  o_h = jnp.concatenate([origins, ones], axis=-1)
  o_loc = jnp.einsum('bnk,bkj->bnj', o_h, world2local)
  o_loc = o_loc[..., :3] / o_loc[..., 3:4]
  d_loc = jnp.einsum('bnk,bkj->bnj', directions, world2local[:, :3, :3])
  pts = o_loc[:, :, None, :] + d_loc[:, :, None, :] * lengths[..., None]
  px = pts[..., 0].reshape(-1)
  py = pts[..., 1].reshape(-1)
  pz = pts[..., 2].reshape(-1)

  vol = jnp.concatenate([densities, features], axis=1).reshape(B, 9, DHW)
  tab = jnp.pad(vol.transpose(0, 2, 1), ((0, 0), (0, 0), (0, 7)))
  tab = tab.reshape(B * DHW, 16)

  mesh = plsc.VectorSubcoreMesh(core_axis_name="c", subcore_axis_name="s")
  run = pl.kernel(
      _interp_body,
      out_type=(jax.ShapeDtypeStruct((N,), jnp.float32),
                jax.ShapeDtypeStruct((N * 8,), jnp.float32)),
      mesh=mesh,
      scratch_types=(
          pltpu.VMEM((CHK,), jnp.float32),
          pltpu.VMEM((CHK,), jnp.float32),
          pltpu.VMEM((CHK,), jnp.float32),
          pltpu.VMEM((NCORN, CHK), jnp.int32),
          pltpu.VMEM((NCORN, CHK), jnp.float32),
          pltpu.VMEM((NCORN * CHK, 16), jnp.float32),
          pltpu.VMEM((CHK,), jnp.float32),
          pltpu.VMEM((CHK * 8,), jnp.float32),
          pltpu.SemaphoreType.DMA,
      ),
      compiler_params=pltpu.CompilerParams(needs_layout_passes=False,
                                           use_tc_tiling_on_sc=False),
  )
  dens_flat, feat_flat = run(tab, px, py, pz)
  return (dens_flat.reshape(B, NR, P, 1), feat_flat.reshape(B, NR, P, 8))
